# Initial kernel scaffold; baseline (speedup 1.0000x reference)
import jax
import jax.numpy as jnp
from jax.experimental import pallas as pl


def kernel(input):
    raise NotImplementedError("write your pallas kernel here")


# speedup vs baseline: 4.1000x; 4.1000x over previous
import functools

import jax
import jax.numpy as jnp
from jax import lax
from jax.experimental import pallas as pl
from jax.experimental.pallas import tpu as pltpu
from jax.experimental.pallas import tpu_sc as plsc

_ROWS = 64
_N = 4096
_LANES = 16
_NCHUNK = _N // _LANES
_NWORKERS = 32
_ROWS_PER_W = _ROWS // _NWORKERS
_NBISECT = 30

_mesh = plsc.VectorSubcoreMesh(core_axis_name="c", subcore_axis_name="s")


@functools.partial(
    pl.kernel,
    mesh=_mesh,
    compiler_params=pltpu.CompilerParams(needs_layout_passes=False),
    out_type=jax.ShapeDtypeStruct((_ROWS * _N,), jnp.float32),
    scratch_types=[
        pltpu.VMEM((_N,), jnp.float32),
        pltpu.VMEM((_N + _LANES,), jnp.float32),
        pltpu.VMEM((_N,), jnp.float32),
    ],
)
def _sparsemax_sc(x_hbm, out_hbm, row_v, comp_v, out_v):
    wid = lax.axis_index("s") * 2 + lax.axis_index("c")

    lane = lax.iota(jnp.int32, _LANES)

    def process_row(row):
        base = row * _N
        pltpu.sync_copy(x_hbm.at[pl.ds(base, _N)], row_v)

        def max_body(i, acc):
            return jnp.maximum(acc, row_v[pl.ds(i * _LANES, _LANES)])

        acc = lax.fori_loop(0, _NCHUNK, max_body, row_v[pl.ds(0, _LANES)])
        mx = jnp.max(acc)

        def comp_body(i, off):
            y = row_v[pl.ds(i * _LANES, _LANES)] - mx
            m = y > -1.0
            plsc.store_compressed(comp_v.at[pl.ds(off, _LANES)], y, mask=m)
            return off + jnp.sum(m.astype(jnp.int32))

        mcount = lax.fori_loop(0, _NCHUNK, comp_body, jnp.int32(0))
        nch = lax.shift_right_logical(mcount + (_LANES - 1), 4)

        def masked_pass(t):
            def body(i, carry):
                ssum, cnt = carry
                v = comp_v[pl.ds(i * _LANES, _LANES)]
                valid = (lane + i * _LANES < mcount) & (v > t)
                ssum = ssum + jnp.where(valid, v, 0.0)
                cnt = cnt + jnp.where(valid, 1, 0)
                return ssum, cnt

            ssum, cnt = lax.fori_loop(
                0, nch, body,
                (jnp.zeros((_LANES,), jnp.float32),
                 jnp.zeros((_LANES,), jnp.int32)))
            return jnp.sum(ssum), jnp.sum(cnt)

        def bis_body(_, lohi):
            lo, hi = lohi
            mid = 0.5 * (lo + hi)
            s, k = masked_pass(mid)
            fmid = s - mid * k.astype(jnp.float32) - 1.0
            ge = fmid >= 0.0
            return jnp.where(ge, mid, lo), jnp.where(ge, hi, mid)

        lo, hi = lax.fori_loop(
            0, _NBISECT, bis_body, (jnp.float32(-1.0), jnp.float32(0.0)))
        mid = 0.5 * (lo + hi)

        s, k = masked_pass(mid)
        num = jnp.full((_LANES,), s - 1.0, jnp.float32)
        den = jnp.full((_LANES,), k, jnp.int32).astype(jnp.float32)
        tau = num / den

        def out_body(i, carry):
            y = row_v[pl.ds(i * _LANES, _LANES)] - mx
            out_v[pl.ds(i * _LANES, _LANES)] = jnp.maximum(y - tau, 0.0)
            return carry

        lax.fori_loop(0, _NCHUNK, out_body, jnp.int32(0))
        pltpu.sync_copy(out_v, out_hbm.at[pl.ds(base, _N)])

    for r in range(_ROWS_PER_W):
        process_row(wid * _ROWS_PER_W + r)


def kernel(input):
    flat = input.reshape(_ROWS * _N)
    out = _sparsemax_sc(flat)
    return out.reshape(_ROWS, _N)


# speedup vs baseline: 5.3433x; 1.3032x over previous
import functools

import jax
import jax.numpy as jnp
from jax import lax
from jax.experimental import pallas as pl
from jax.experimental.pallas import tpu as pltpu
from jax.experimental.pallas import tpu_sc as plsc

_ROWS = 64
_N = 4096
_LANES = 16
_NCHUNK = _N // _LANES
_UNROLL = 8
_NGROUP = _NCHUNK // _UNROLL
_NWORKERS = 32
_ROWS_PER_W = _ROWS // _NWORKERS
_NBISECT = 22
_SENTINEL = -1e30

_mesh = plsc.VectorSubcoreMesh(core_axis_name="c", subcore_axis_name="s")


@functools.partial(
    pl.kernel,
    mesh=_mesh,
    compiler_params=pltpu.CompilerParams(needs_layout_passes=False),
    out_type=jax.ShapeDtypeStruct((_ROWS * _N,), jnp.float32),
    scratch_types=[
        pltpu.VMEM((_N,), jnp.float32),
        pltpu.VMEM((_N,), jnp.float32),
        pltpu.VMEM((_N + _LANES,), jnp.float32),
        pltpu.VMEM((_N,), jnp.float32),
        pltpu.VMEM((_N,), jnp.float32),
        pltpu.SemaphoreType.DMA,
        pltpu.SemaphoreType.DMA,
        pltpu.SemaphoreType.DMA,
        pltpu.SemaphoreType.DMA,
    ],
)
def _sparsemax_sc(x_hbm, out_hbm, row_v0, row_v1, comp_v, out_v0, out_v1,
                  ld_sem0, ld_sem1, st_sem0, st_sem1):
    wid = lax.axis_index("s") * 2 + lax.axis_index("c")
    base0 = wid * (_ROWS_PER_W * _N)
    base1 = base0 + _N

    ld0 = pltpu.make_async_copy(x_hbm.at[pl.ds(base0, _N)], row_v0, ld_sem0)
    ld1 = pltpu.make_async_copy(x_hbm.at[pl.ds(base1, _N)], row_v1, ld_sem1)
    ld0.start()
    ld1.start()

    def process_row(row_v, out_v):
        def max_body(i, accs):
            a0, a1 = accs
            b = i * (_UNROLL * _LANES)
            vs = [row_v[pl.ds(b + j * _LANES, _LANES)] for j in range(_UNROLL)]
            a0 = jnp.maximum(a0, jnp.maximum(jnp.maximum(vs[0], vs[1]),
                                             jnp.maximum(vs[2], vs[3])))
            a1 = jnp.maximum(a1, jnp.maximum(jnp.maximum(vs[4], vs[5]),
                                             jnp.maximum(vs[6], vs[7])))
            return a0, a1

        init = jnp.full((_LANES,), _SENTINEL, jnp.float32)
        a0, a1 = lax.fori_loop(0, _NGROUP, max_body, (init, init))
        mx = jnp.max(jnp.maximum(a0, a1))
        mxv = jnp.full((_LANES,), mx, jnp.float32)

        def comp_body(i, off):
            b = i * (_UNROLL * _LANES)
            ys = [row_v[pl.ds(b + j * _LANES, _LANES)] - mxv
                  for j in range(_UNROLL)]
            ms = [y > -1.0 for y in ys]
            pcs = [plsc.all_reduce_population_count(m)[0] for m in ms]
            offs = []
            for j in range(_UNROLL):
                offs.append(off)
                off = off + pcs[j]
            for j in range(_UNROLL):
                plsc.store_compressed(comp_v.at[pl.ds(offs[j], _LANES)],
                                      ys[j], mask=ms[j])
            return off

        mcount = lax.fori_loop(0, _NGROUP, comp_body, jnp.int32(0))
        comp_v[pl.ds(mcount, _LANES)] = init
        nch = lax.shift_right_logical(mcount + (_LANES - 1), 4)

        def bis_body(_, lohi):
            lo, hi = lohi
            mid = 0.5 * (lo + hi)
            midv = jnp.full((_LANES,), mid, jnp.float32)

            def body(i, ssum):
                v = comp_v[pl.ds(i * _LANES, _LANES)]
                return ssum + jnp.maximum(v - midv, 0.0)

            ssum = lax.fori_loop(0, nch, body,
                                 jnp.zeros((_LANES,), jnp.float32))
            ge = jnp.sum(ssum) >= 1.0
            return jnp.where(ge, mid, lo), jnp.where(ge, hi, mid)

        lo, hi = lax.fori_loop(
            0, _NBISECT, bis_body, (jnp.float32(-1.0), jnp.float32(0.0)))
        tauv = jnp.full((_LANES,), 0.5 * (lo + hi), jnp.float32)

        for _ in range(2):
            def corr_body(i, carry):
                ssum, cnt = carry
                v = comp_v[pl.ds(i * _LANES, _LANES)]
                act = v > tauv
                ssum = ssum + jnp.where(act, v, 0.0)
                cnt = cnt + jnp.where(act, 1, 0)
                return ssum, cnt

            ssum, cnt = lax.fori_loop(
                0, nch, corr_body,
                (jnp.zeros((_LANES,), jnp.float32),
                 jnp.zeros((_LANES,), jnp.int32)))
            num = jnp.full((_LANES,), jnp.sum(ssum) - 1.0, jnp.float32)
            den = jnp.full((_LANES,), jnp.sum(cnt), jnp.int32)
            tauv = num / den.astype(jnp.float32)

        cv = mxv + tauv

        def out_body(i, carry):
            b = i * (_UNROLL * _LANES)
            for j in range(_UNROLL):
                s = pl.ds(b + j * _LANES, _LANES)
                out_v[s] = jnp.maximum(row_v[s] - cv, 0.0)
            return carry

        lax.fori_loop(0, _NGROUP, out_body, jnp.int32(0))

    ld0.wait()
    process_row(row_v0, out_v0)
    st0 = pltpu.make_async_copy(out_v0, out_hbm.at[pl.ds(base0, _N)], st_sem0)
    st0.start()
    ld1.wait()
    process_row(row_v1, out_v1)
    st1 = pltpu.make_async_copy(out_v1, out_hbm.at[pl.ds(base1, _N)], st_sem1)
    st1.start()
    st0.wait()
    st1.wait()


def kernel(input):
    flat = input.reshape(_ROWS * _N)
    out = _sparsemax_sc(flat)
    return out.reshape(_ROWS, _N)


# speedup vs baseline: 5.8695x; 1.0985x over previous
import functools

import jax
import jax.numpy as jnp
from jax import lax
from jax.experimental import pallas as pl
from jax.experimental.pallas import tpu as pltpu
from jax.experimental.pallas import tpu_sc as plsc

_ROWS = 64
_N = 4096
_LANES = 16
_NCHUNK = _N // _LANES
_UNROLL = 8
_NGROUP = _NCHUNK // _UNROLL
_NWORKERS = 32
_ROWS_PER_W = _ROWS // _NWORKERS
_NBISECT = 22
_SENTINEL = -1e30

_mesh = plsc.VectorSubcoreMesh(core_axis_name="c", subcore_axis_name="s")


@functools.partial(
    pl.kernel,
    mesh=_mesh,
    compiler_params=pltpu.CompilerParams(needs_layout_passes=False,
                                         use_tc_tiling_on_sc=True),
    out_type=jax.ShapeDtypeStruct((_ROWS, _N), jnp.float32),
    scratch_types=[
        pltpu.VMEM((_N,), jnp.float32),
        pltpu.VMEM((_N,), jnp.float32),
        pltpu.VMEM((_N + _LANES,), jnp.float32),
        pltpu.VMEM((_N,), jnp.float32),
        pltpu.VMEM((_N,), jnp.float32),
        pltpu.SemaphoreType.DMA,
        pltpu.SemaphoreType.DMA,
        pltpu.SemaphoreType.DMA,
        pltpu.SemaphoreType.DMA,
    ],
)
def _sparsemax_sc(x_hbm, out_hbm, row_v0, row_v1, comp_v, out_v0, out_v1,
                  ld_sem0, ld_sem1, st_sem0, st_sem1):
    wid = lax.axis_index("s") * 2 + lax.axis_index("c")
    row0 = wid * _ROWS_PER_W
    row1 = row0 + 1

    ld0 = pltpu.make_async_copy(x_hbm.at[row0], row_v0, ld_sem0)
    ld1 = pltpu.make_async_copy(x_hbm.at[row1], row_v1, ld_sem1)
    ld0.start()
    ld1.start()

    def process_row(row_v, out_v):
        def max_body(i, accs):
            a0, a1 = accs
            b = i * (_UNROLL * _LANES)
            vs = [row_v[pl.ds(b + j * _LANES, _LANES)] for j in range(_UNROLL)]
            a0 = jnp.maximum(a0, jnp.maximum(jnp.maximum(vs[0], vs[1]),
                                             jnp.maximum(vs[2], vs[3])))
            a1 = jnp.maximum(a1, jnp.maximum(jnp.maximum(vs[4], vs[5]),
                                             jnp.maximum(vs[6], vs[7])))
            return a0, a1

        init = jnp.full((_LANES,), _SENTINEL, jnp.float32)
        a0, a1 = lax.fori_loop(0, _NGROUP, max_body, (init, init))
        mx = jnp.max(jnp.maximum(a0, a1))
        mxv = jnp.full((_LANES,), mx, jnp.float32)

        def comp_body(i, off):
            b = i * (_UNROLL * _LANES)
            ys = [row_v[pl.ds(b + j * _LANES, _LANES)] - mxv
                  for j in range(_UNROLL)]
            ms = [y > -1.0 for y in ys]
            pcs = [plsc.all_reduce_population_count(m)[0] for m in ms]
            offs = []
            for j in range(_UNROLL):
                offs.append(off)
                off = off + pcs[j]
            for j in range(_UNROLL):
                plsc.store_compressed(comp_v.at[pl.ds(offs[j], _LANES)],
                                      ys[j], mask=ms[j])
            return off

        mcount = lax.fori_loop(0, _NGROUP, comp_body, jnp.int32(0))
        comp_v[pl.ds(mcount, _LANES)] = init
        nch = lax.shift_right_logical(mcount + (_LANES - 1), 4)

        def bis_body(_, lohi):
            lo, hi = lohi
            mid = 0.5 * (lo + hi)
            midv = jnp.full((_LANES,), mid, jnp.float32)

            def body(i, ssum):
                v = comp_v[pl.ds(i * _LANES, _LANES)]
                return ssum + jnp.maximum(v - midv, 0.0)

            ssum = lax.fori_loop(0, nch, body,
                                 jnp.zeros((_LANES,), jnp.float32))
            ge = jnp.sum(ssum) >= 1.0
            return jnp.where(ge, mid, lo), jnp.where(ge, hi, mid)

        lo, hi = lax.fori_loop(
            0, _NBISECT, bis_body, (jnp.float32(-1.0), jnp.float32(0.0)))
        tauv = jnp.full((_LANES,), 0.5 * (lo + hi), jnp.float32)

        for _ in range(2):
            def corr_body(i, carry):
                ssum, cnt = carry
                v = comp_v[pl.ds(i * _LANES, _LANES)]
                act = v > tauv
                ssum = ssum + jnp.where(act, v, 0.0)
                cnt = cnt + jnp.where(act, 1, 0)
                return ssum, cnt

            ssum, cnt = lax.fori_loop(
                0, nch, corr_body,
                (jnp.zeros((_LANES,), jnp.float32),
                 jnp.zeros((_LANES,), jnp.int32)))
            num = jnp.full((_LANES,), jnp.sum(ssum) - 1.0, jnp.float32)
            den = jnp.full((_LANES,), jnp.sum(cnt), jnp.int32)
            tauv = num / den.astype(jnp.float32)

        cv = mxv + tauv

        def out_body(i, carry):
            b = i * (_UNROLL * _LANES)
            for j in range(_UNROLL):
                s = pl.ds(b + j * _LANES, _LANES)
                out_v[s] = jnp.maximum(row_v[s] - cv, 0.0)
            return carry

        lax.fori_loop(0, _NGROUP, out_body, jnp.int32(0))

    ld0.wait()
    process_row(row_v0, out_v0)
    st0 = pltpu.make_async_copy(out_v0, out_hbm.at[row0], st_sem0)
    st0.start()
    ld1.wait()
    process_row(row_v1, out_v1)
    st1 = pltpu.make_async_copy(out_v1, out_hbm.at[row1], st_sem1)
    st1.start()
    st0.wait()
    st1.wait()


def kernel(input):
    return _sparsemax_sc(input)
